# baseline (speedup 1.0000x reference)
import functools

import jax
import jax.numpy as jnp
from jax import lax
from jax.experimental import pallas as pl
from jax.experimental.pallas import tpu as pltpu
from jax.experimental.pallas import tpu_sc as plsc

B = 256
N = 256
EPG = 512
IN_C = 5
HID = 512
STEPS = 4
G = 8


_SC = plsc.get_sparse_core_info()
_NW = _SC.num_cores * _SC.num_subcores
_TOT = B * EPG
_CHUNK = _TOT // _NW
_VL = 16


def _sc_count(src_hbm, dst_hbm, out_hbm, s_v, d_v, acc_v):
    wid = lax.axis_index("s") * _SC.num_cores + lax.axis_index("c")
    base = wid * _CHUNK
    pltpu.sync_copy(src_hbm.at[pl.ds(base, _CHUNK)], s_v)
    pltpu.sync_copy(dst_hbm.at[pl.ds(base, _CHUNK)], d_v)

    def body(i, acc):
        sv = s_v[pl.ds(i * _VL, _VL)]
        dv = d_v[pl.ds(i * _VL, _VL)]
        return acc + jnp.where(sv == dv, 1.0, 0.0).astype(jnp.float32)

    acc = lax.fori_loop(0, _CHUNK // _VL, body,
                        jnp.zeros((_VL,), jnp.float32))
    acc_v[...] = acc
    pltpu.sync_copy(acc_v, out_hbm.at[wid])


def _main_kernel(cnt_ref, xT_ref, srow_ref, drow_ref, scol_ref, dcol_ref,
                 fdx_ref, nats_ref,
                 lin_bf_ref, linb_ref, g1_bf_ref, g1b_ref, g2_bf_ref, g2b_ref,
                 flT_ref, flb_ref, m1T_ref, m1b_ref, m2T_ref, m2b_ref,
                 m3T_ref, m3b_ref, out_ref):
    p = pl.program_id(0)
    x = jnp.concatenate([xT_ref[g] for g in range(G)], axis=1)

    n_iota_r = jax.lax.broadcasted_iota(jnp.int32, (N, EPG), 0)
    n_iota_c = jax.lax.broadcasted_iota(jnp.int32, (EPG, N), 1)
    ir = jax.lax.broadcasted_iota(jnp.int32, (N, N), 0)
    ic = jax.lax.broadcasted_iota(jnp.int32, (N, N), 1)
    eye = jnp.where(ir == ic, 1.0, 0.0)
    STs, Ds = [], []
    for g in range(G):
        srow = srow_ref[g]
        drow = drow_ref[g]
        scol = scol_ref[g]
        dcol = dcol_ref[g]
        ST = jnp.where((n_iota_r == srow) & (srow != drow), 1.0, 0.0)
        D = jnp.where((n_iota_c == dcol) & (scol != dcol), 1.0, 0.0)
        c_tot = jnp.sum(jnp.sum(cnt_ref[...], axis=1, keepdims=True),
                        axis=0, keepdims=True)
        c_extra = jnp.where((p == 0) & (g == 0), c_tot, 0.0)
        eye_d = eye + jnp.where((ir == 0) & (ic == 0), c_extra, 0.0)
        STs.append(jnp.concatenate([ST, eye], axis=1))
        Ds.append(jnp.concatenate([D, eye_d], axis=0))

    def prop(v):
        outs = []
        for g in range(G):
            vg = v[:, g * N:(g + 1) * N]
            tmp = jnp.dot(vg, STs[g], preferred_element_type=jnp.float32,
                    precision=jax.lax.Precision.HIGHEST)
            outs.append(jnp.dot(tmp, Ds[g],
                    preferred_element_type=jnp.float32,
                    precision=jax.lax.Precision.HIGHEST))
        return jnp.concatenate(outs, axis=1)

    for gc in range(STEPS):
        if gc > 0:
            h = jnp.tanh(jnp.dot(lin_bf_ref[gc], x.astype(jnp.bfloat16),
                                 preferred_element_type=jnp.float32)
                         + linb_ref[gc])
            h = jnp.tanh(jnp.dot(g1_bf_ref[gc], h.astype(jnp.bfloat16),
                                 preferred_element_type=jnp.float32)
                         + g1b_ref[gc])
            x = jnp.dot(g2_bf_ref[gc], h.astype(jnp.bfloat16),
                        preferred_element_type=jnp.float32) + g2b_ref[gc]
        x = prop(x)

    lane = jax.lax.broadcasted_iota(jnp.int32, (1, N), 1)
    ms = []
    for g in range(G):
        xg = x[:, g * N:(g + 1) * N]
        mask = (lane < nats_ref[g]) & (fdx_ref[g] == 1)
        m = jnp.max(jnp.where(mask, xg, -jnp.inf), axis=1, keepdims=True)
        ms.append(jnp.where(jnp.isfinite(m), m, 0.0))
    m = jnp.concatenate(ms, axis=1)

    h = jnp.tanh(jnp.dot(flT_ref[...], m.astype(jnp.bfloat16),
                         preferred_element_type=jnp.float32) + flb_ref[...])
    h = jnp.tanh(jnp.dot(m1T_ref[...], h.astype(jnp.bfloat16),
                         preferred_element_type=jnp.float32) + m1b_ref[...])
    h = jnp.tanh(jnp.dot(m2T_ref[...], h.astype(jnp.bfloat16),
                         preferred_element_type=jnp.float32) + m2b_ref[...])
    o = jnp.dot(m3T_ref[...], h.astype(jnp.bfloat16),
                preferred_element_type=jnp.float32) + m3b_ref[...]
    out_ref[0] = o


def _full_spec(shape):
    nd = len(shape)
    return pl.BlockSpec(shape, lambda p, _nd=nd: (0,) * _nd)


def kernel(inputs, labels, rval, findex, nats, lin_W, lin_b, g1_W, g1_b,
           g2_W, g2_b, fl_W, fl_b, m1_W, m1_b, m2_W, m2_b, m3_W, m3_b):
    src = labels[:, :, 0]
    dst = labels[:, :, 1]
    srow = src.reshape(B, 1, EPG)
    drow = dst.reshape(B, 1, EPG)
    scol = src.reshape(B, EPG, 1)
    dcol = dst.reshape(B, EPG, 1)
    xT = inputs.transpose(0, 2, 1)
    fdx = findex[:, :, 0].reshape(B, 1, N)
    natsr = nats.reshape(B, 1, 1)

    lin_bf = lin_W.transpose(0, 2, 1).astype(jnp.bfloat16)
    linb = lin_b[:, :, None]
    g1_bf = g1_W.transpose(0, 2, 1).astype(jnp.bfloat16)
    g1b = g1_b[:, :, None]
    g2_bf = g2_W.transpose(0, 2, 1).astype(jnp.bfloat16)
    g2b = g2_b[:, :, None]
    flT = fl_W.T.astype(jnp.bfloat16)
    flb = fl_b[:, None]
    m1T = m1_W.T.astype(jnp.bfloat16)
    m1b = m1_b[:, None]
    m2T = m2_W.T.astype(jnp.bfloat16)
    m2b = m2_b[:, None]
    m3T = m3_W.T.astype(jnp.bfloat16)
    m3b = m3_b[:, None]

    sc_count = functools.partial(
        pl.kernel, _sc_count,
        mesh=plsc.VectorSubcoreMesh(core_axis_name="c", subcore_axis_name="s"),
        out_type=jax.ShapeDtypeStruct((_NW, _VL), jnp.float32),
        scratch_types=[
            pltpu.VMEM((_CHUNK,), jnp.int32),
            pltpu.VMEM((_CHUNK,), jnp.int32),
            pltpu.VMEM((_VL,), jnp.float32),
        ],
    )
    cnt = sc_count()(src.reshape(_TOT), dst.reshape(_TOT))

    grid = (B // G,)
    in_specs = [
        _full_spec((_NW, _VL)),
        pl.BlockSpec((G, IN_C, N), lambda p: (p, 0, 0)),
        pl.BlockSpec((G, 1, EPG), lambda p: (p, 0, 0)),
        pl.BlockSpec((G, 1, EPG), lambda p: (p, 0, 0)),
        pl.BlockSpec((G, EPG, 1), lambda p: (p, 0, 0)),
        pl.BlockSpec((G, EPG, 1), lambda p: (p, 0, 0)),
        pl.BlockSpec((G, 1, N), lambda p: (p, 0, 0)),
        pl.BlockSpec((G, 1, 1), lambda p: (p, 0, 0)),
        _full_spec((STEPS, HID, IN_C)),
        _full_spec((STEPS, HID, 1)),
        _full_spec((STEPS, HID, HID)),
        _full_spec((STEPS, HID, 1)),
        _full_spec((STEPS, IN_C, HID)),
        _full_spec((STEPS, IN_C, 1)),
        _full_spec((64, IN_C)),
        _full_spec((64, 1)),
        _full_spec((32, 64)),
        _full_spec((32, 1)),
        _full_spec((16, 32)),
        _full_spec((16, 1)),
        _full_spec((1, 16)),
        _full_spec((1, 1)),
    ]
    out = pl.pallas_call(
        _main_kernel,
        grid=grid,
        in_specs=in_specs,
        out_specs=pl.BlockSpec((1, 1, G), lambda p: (p, 0, 0)),
        out_shape=jax.ShapeDtypeStruct((B // G, 1, G), jnp.float32),
        compiler_params=pltpu.CompilerParams(
            dimension_semantics=("arbitrary",),
        ),
    )(cnt, xT, srow, drow, scol, dcol, fdx, natsr,
      lin_bf, linb, g1_bf, g1b, g2_bf, g2b,
      flT, flb, m1T, m1b, m2T, m2b, m3T, m3b)
    return out.reshape(B, 1)
